# baseline (speedup 1.0000x reference)
import jax
import jax.numpy as jnp
from jax import lax
from jax.experimental import pallas as pl
from jax.experimental.pallas import tpu as pltpu
from jax.experimental.pallas import tpu_sc as plsc

_N_SPLATS = 50000
_M = 4096
_BS = 512
_NB = 98
_NP = _NB * _BS
_MC = 512
_NMC = _M // _MC
_MAX_DISTANCE = 1.0
_LANDMARK_WEIGHT = 1.0

_NC = 2
_NS = 16
_NW = _NC * _NS
_LPW = _M // _NW
_LANES = 16


def _min_kernel(poset_ref, lm4_ref, x_ref, camplus_ref, ids_ref,
                camb_ref, y2_ref, run_ref):
    j = pl.program_id(0)
    i = pl.program_id(1)
    msl = pl.ds(j * _MC, _MC)

    @pl.when(i == 0)
    def _init():
        cam = lax.dot_general(lm4_ref[...], poset_ref[...],
                              (((1,), (0,)), ((), ())),
                              preferred_element_type=jnp.float32)
        camplus_ref[msl, 0:3] = cam
        y2_ref[...] = jnp.sum(cam * cam, axis=1, keepdims=True)
        camb_ref[...] = (-2.0 * cam).astype(jnp.bfloat16)
        run_ref[...] = jnp.full((_MC, _BS), jnp.int32(0x7FFFFFFF), jnp.int32)

    xb = x_ref[0]
    x2 = (xb[0:1, :] * xb[0:1, :] + xb[1:2, :] * xb[1:2, :]
          + xb[2:3, :] * xb[2:3, :])
    mm = lax.dot_general(camb_ref[...], xb.astype(jnp.bfloat16),
                         (((1,), (0,)), ((), ())),
                         preferred_element_type=jnp.float32)
    d2 = (x2 + y2_ref[...]) + mm
    d2 = jnp.maximum(d2, 1e-12)
    key = (lax.bitcast_convert_type(d2, jnp.int32) & jnp.int32(-128)) | i
    run_ref[...] = jnp.minimum(run_ref[...], key)

    @pl.when(i == _NB - 1)
    def _extract():
        keys = run_ref[...]
        bi = keys & jnp.int32(127)
        d2t = lax.bitcast_convert_type(keys & jnp.int32(-128), jnp.float32)
        bm = jnp.min(d2t, axis=1, keepdims=True)
        lane = lax.broadcasted_iota(jnp.int32, (_MC, _BS), 1)
        gid = bi * _BS + lane
        cand = jnp.where(d2t == bm, gid, jnp.int32(2 ** 30))
        ids_ref[msl, :] = jnp.min(cand, axis=1, keepdims=True)
        camplus_ref[msl, 3:4] = bm


def _sc_loss_kernel(splat16_ref, camplus_ref, ids_ref, psum_ref, pcnt_ref,
                    idx_v, rows_v, cam_v, acc_v, sem):
    cid = lax.axis_index("c")
    sid = lax.axis_index("s")
    w = sid * _NC + cid
    base = w * _LPW
    pltpu.sync_copy(ids_ref.at[pl.ds(base, _LPW)], idx_v)
    pltpu.async_copy(splat16_ref.at[idx_v], rows_v, sem).wait()
    pltpu.sync_copy(camplus_ref.at[pl.ds(base, _LPW)], cam_v)
    lanes = lax.iota(jnp.int32, 16)
    zero = jnp.zeros((16,), jnp.int32)
    acc_s = jnp.zeros((16,), jnp.float32)
    acc_c = jnp.zeros((16,), jnp.float32)
    for c8 in range(_LPW // _LANES):
        ridx = lanes + c8 * _LANES
        gx = plsc.load_gather(rows_v, [ridx, zero])
        gy = plsc.load_gather(rows_v, [ridx, zero + 1])
        gz = plsc.load_gather(rows_v, [ridx, zero + 2])
        cx = plsc.load_gather(cam_v, [ridx, zero])
        cy = plsc.load_gather(cam_v, [ridx, zero + 1])
        cz = plsc.load_gather(cam_v, [ridx, zero + 2])
        md = plsc.load_gather(cam_v, [ridx, zero + 3])
        dx = gx - cx
        dy = gy - cy
        dz = gz - cz
        sq = dx * dx + dy * dy + dz * dz
        valid = md < (_MAX_DISTANCE * _MAX_DISTANCE)
        acc_s = acc_s + jnp.where(valid, sq, 0.0)
        acc_c = acc_c + jnp.where(valid, 1.0, 0.0)
    acc_v[...] = acc_s
    pltpu.sync_copy(acc_v, psum_ref.at[w])
    acc_v[...] = acc_c
    pltpu.sync_copy(acc_v, pcnt_ref.at[w])


def _finish_kernel(psum_ref, pcnt_ref, out_ref):
    s = jnp.sum(psum_ref[...])
    nv = jnp.sum(pcnt_ref[...])
    out_ref[0, 0] = _LANDMARK_WEIGHT * s / jnp.maximum(3.0 * nv, 1.0)


def kernel(splat_positions, camera_pose, landmarks_3d, landmarks_2d,
           camera_intrinsics):
    del landmarks_2d, camera_intrinsics
    xp = jnp.pad(splat_positions, ((0, _NP - _N_SPLATS), (0, 0)),
                 constant_values=1e6)
    x = xp.reshape(_NB, _BS, 3).transpose(0, 2, 1)
    poset = camera_pose[:3, :].T.astype(jnp.bfloat16)
    lm4 = jnp.concatenate(
        [landmarks_3d, jnp.ones((_M, 1), jnp.float32)], axis=1
    ).astype(jnp.bfloat16)
    splat16 = jnp.pad(splat_positions, ((0, 0), (0, 13)))

    camplus, ids = pl.pallas_call(
        _min_kernel,
        grid=(_NMC, _NB),
        in_specs=[
            pl.BlockSpec((4, 3), lambda j, i: (0, 0)),
            pl.BlockSpec((_MC, 4), lambda j, i: (j, 0)),
            pl.BlockSpec((1, 3, _BS), lambda j, i: (i, 0, 0)),
        ],
        out_specs=[
            pl.BlockSpec((_M, 16), lambda j, i: (0, 0)),
            pl.BlockSpec((_M, 1), lambda j, i: (0, 0)),
        ],
        out_shape=[
            jax.ShapeDtypeStruct((_M, 16), jnp.float32),
            jax.ShapeDtypeStruct((_M, 1), jnp.int32),
        ],
        scratch_shapes=[
            pltpu.VMEM((_MC, 3), jnp.bfloat16),
            pltpu.VMEM((_MC, 1), jnp.float32),
            pltpu.VMEM((_MC, _BS), jnp.int32),
        ],
    )(poset, lm4, x)

    sc_loss = pl.kernel(
        _sc_loss_kernel,
        out_type=[
            jax.ShapeDtypeStruct((_NW, 16), jnp.float32),
            jax.ShapeDtypeStruct((_NW, 16), jnp.float32),
        ],
        mesh=plsc.VectorSubcoreMesh(core_axis_name="c", subcore_axis_name="s",
                                    num_cores=_NC, num_subcores=_NS),
        compiler_params=pltpu.CompilerParams(needs_layout_passes=False,
                                             use_tc_tiling_on_sc=False),
        scratch_types=[
            pltpu.VMEM((_LPW,), jnp.int32),
            pltpu.VMEM((_LPW, 16), jnp.float32),
            pltpu.VMEM((_LPW, 16), jnp.float32),
            pltpu.VMEM((16,), jnp.float32),
            pltpu.SemaphoreType.DMA,
        ],
    )
    psum, pcnt = sc_loss(splat16, camplus, ids.reshape(_M))

    out = pl.pallas_call(
        _finish_kernel,
        in_specs=[
            pl.BlockSpec(memory_space=pltpu.VMEM),
            pl.BlockSpec(memory_space=pltpu.VMEM),
        ],
        out_specs=pl.BlockSpec(memory_space=pltpu.SMEM),
        out_shape=jax.ShapeDtypeStruct((1, 1), jnp.float32),
    )(psum, pcnt)
    return out[0, 0]


# speedup vs baseline: 1.8370x; 1.8370x over previous
import jax
import jax.numpy as jnp
from jax import lax
from jax.experimental import pallas as pl
from jax.experimental.pallas import tpu as pltpu
from jax.experimental.pallas import tpu_sc as plsc

_N_SPLATS = 50000
_M = 4096
_BS = 512
_NB = 98
_NP = _NB * _BS
_MC = 4096
_NMC = _M // _MC
_MAX_DISTANCE = 1.0
_LANDMARK_WEIGHT = 1.0

_NC = 2
_NS = 16
_NW = _NC * _NS
_LPW = _M // _NW
_LANES = 16


def _min_kernel(poset_ref, lm4_ref, x_ref, camplus_ref, ids_ref,
                camb_ref, y2_ref, run_ref):
    j = pl.program_id(0)
    i = pl.program_id(1)
    msl = pl.ds(j * _MC, _MC)

    @pl.when(i == 0)
    def _init():
        cam = lax.dot_general(lm4_ref[...], poset_ref[...],
                              (((1,), (0,)), ((), ())),
                              preferred_element_type=jnp.float32)
        camplus_ref[msl, 0:3] = cam
        y2_ref[...] = jnp.sum(cam * cam, axis=1, keepdims=True)
        camb_ref[...] = (-2.0 * cam).astype(jnp.bfloat16)
        run_ref[...] = jnp.full((_MC, _BS), jnp.int32(0x7FFFFFFF), jnp.int32)

    xb = x_ref[0]
    x2 = (xb[0:1, :] * xb[0:1, :] + xb[1:2, :] * xb[1:2, :]
          + xb[2:3, :] * xb[2:3, :])
    mm = lax.dot_general(camb_ref[...], xb.astype(jnp.bfloat16),
                         (((1,), (0,)), ((), ())),
                         preferred_element_type=jnp.float32)
    d2 = (x2 + y2_ref[...]) + mm
    d2 = jnp.maximum(d2, 1e-12)
    key = (lax.bitcast_convert_type(d2, jnp.int32) & jnp.int32(-128)) | i
    run_ref[...] = jnp.minimum(run_ref[...], key)

    @pl.when(i == _NB - 1)
    def _extract():
        keys = run_ref[...]
        bi = keys & jnp.int32(127)
        d2t = lax.bitcast_convert_type(keys & jnp.int32(-128), jnp.float32)
        bm = jnp.min(d2t, axis=1, keepdims=True)
        lane = lax.broadcasted_iota(jnp.int32, (_MC, _BS), 1)
        gid = bi * _BS + lane
        cand = jnp.where(d2t == bm, gid, jnp.int32(2 ** 30))
        ids_ref[msl, :] = jnp.min(cand, axis=1, keepdims=True)
        camplus_ref[msl, 3:4] = bm


def _sc_loss_kernel(splat16_ref, camplus_ref, ids_ref, psum_ref, pcnt_ref,
                    idx_v, rows_v, cam_v, acc_v, sem):
    cid = lax.axis_index("c")
    sid = lax.axis_index("s")
    w = sid * _NC + cid
    base = w * _LPW
    pltpu.sync_copy(ids_ref.at[pl.ds(base, _LPW)], idx_v)
    pltpu.async_copy(splat16_ref.at[idx_v], rows_v, sem).wait()
    pltpu.sync_copy(camplus_ref.at[pl.ds(base, _LPW)], cam_v)
    lanes = lax.iota(jnp.int32, 16)
    zero = jnp.zeros((16,), jnp.int32)
    acc_s = jnp.zeros((16,), jnp.float32)
    acc_c = jnp.zeros((16,), jnp.float32)
    for c8 in range(_LPW // _LANES):
        ridx = lanes + c8 * _LANES
        gx = plsc.load_gather(rows_v, [ridx, zero])
        gy = plsc.load_gather(rows_v, [ridx, zero + 1])
        gz = plsc.load_gather(rows_v, [ridx, zero + 2])
        cx = plsc.load_gather(cam_v, [ridx, zero])
        cy = plsc.load_gather(cam_v, [ridx, zero + 1])
        cz = plsc.load_gather(cam_v, [ridx, zero + 2])
        md = plsc.load_gather(cam_v, [ridx, zero + 3])
        dx = gx - cx
        dy = gy - cy
        dz = gz - cz
        sq = dx * dx + dy * dy + dz * dz
        valid = md < (_MAX_DISTANCE * _MAX_DISTANCE)
        acc_s = acc_s + jnp.where(valid, sq, 0.0)
        acc_c = acc_c + jnp.where(valid, 1.0, 0.0)
    acc_v[...] = acc_s
    pltpu.sync_copy(acc_v, psum_ref.at[w])
    acc_v[...] = acc_c
    pltpu.sync_copy(acc_v, pcnt_ref.at[w])


def _finish_kernel(psum_ref, pcnt_ref, out_ref):
    s = jnp.sum(psum_ref[...])
    nv = jnp.sum(pcnt_ref[...])
    out_ref[0, 0] = _LANDMARK_WEIGHT * s / jnp.maximum(3.0 * nv, 1.0)


def kernel(splat_positions, camera_pose, landmarks_3d, landmarks_2d,
           camera_intrinsics):
    del landmarks_2d, camera_intrinsics
    xp = jnp.pad(splat_positions, ((0, _NP - _N_SPLATS), (0, 0)),
                 constant_values=1e6)
    x = xp.reshape(_NB, _BS, 3).transpose(0, 2, 1)
    poset = camera_pose[:3, :].T.astype(jnp.bfloat16)
    lm4 = jnp.concatenate(
        [landmarks_3d, jnp.ones((_M, 1), jnp.float32)], axis=1
    ).astype(jnp.bfloat16)
    splat16 = jnp.pad(splat_positions, ((0, 0), (0, 13)))

    camplus, ids = pl.pallas_call(
        _min_kernel,
        grid=(_NMC, _NB),
        in_specs=[
            pl.BlockSpec((4, 3), lambda j, i: (0, 0)),
            pl.BlockSpec((_MC, 4), lambda j, i: (j, 0)),
            pl.BlockSpec((1, 3, _BS), lambda j, i: (i, 0, 0)),
        ],
        out_specs=[
            pl.BlockSpec((_M, 16), lambda j, i: (0, 0)),
            pl.BlockSpec((_M, 1), lambda j, i: (0, 0)),
        ],
        out_shape=[
            jax.ShapeDtypeStruct((_M, 16), jnp.float32),
            jax.ShapeDtypeStruct((_M, 1), jnp.int32),
        ],
        scratch_shapes=[
            pltpu.VMEM((_MC, 3), jnp.bfloat16),
            pltpu.VMEM((_MC, 1), jnp.float32),
            pltpu.VMEM((_MC, _BS), jnp.int32),
        ],
    )(poset, lm4, x)

    sc_loss = pl.kernel(
        _sc_loss_kernel,
        out_type=[
            jax.ShapeDtypeStruct((_NW, 16), jnp.float32),
            jax.ShapeDtypeStruct((_NW, 16), jnp.float32),
        ],
        mesh=plsc.VectorSubcoreMesh(core_axis_name="c", subcore_axis_name="s",
                                    num_cores=_NC, num_subcores=_NS),
        compiler_params=pltpu.CompilerParams(needs_layout_passes=False,
                                             use_tc_tiling_on_sc=False),
        scratch_types=[
            pltpu.VMEM((_LPW,), jnp.int32),
            pltpu.VMEM((_LPW, 16), jnp.float32),
            pltpu.VMEM((_LPW, 16), jnp.float32),
            pltpu.VMEM((16,), jnp.float32),
            pltpu.SemaphoreType.DMA,
        ],
    )
    psum, pcnt = sc_loss(splat16, camplus, ids.reshape(_M))

    out = pl.pallas_call(
        _finish_kernel,
        in_specs=[
            pl.BlockSpec(memory_space=pltpu.VMEM),
            pl.BlockSpec(memory_space=pltpu.VMEM),
        ],
        out_specs=pl.BlockSpec(memory_space=pltpu.SMEM),
        out_shape=jax.ShapeDtypeStruct((1, 1), jnp.float32),
    )(psum, pcnt)
    return out[0, 0]


# speedup vs baseline: 2.3422x; 1.2751x over previous
import jax
import jax.numpy as jnp
from jax import lax
from jax.experimental import pallas as pl
from jax.experimental.pallas import tpu as pltpu
from jax.experimental.pallas import tpu_sc as plsc

_N_SPLATS = 50000
_M = 4096
_BS = 512
_NB = 98
_NP = _NB * _BS
_MC = 4096
_NMC = _M // _MC
_MAX_DISTANCE = 1.0
_LANDMARK_WEIGHT = 1.0

_NC = 2
_NS = 16
_NW = _NC * _NS
_LPW = _M // _NW
_LANES = 16


def _split3(v, lo_ref, sl):
    lo_ref[:, sl:sl + 1] = v.astype(jnp.bfloat16)
    r1 = v - lo_ref[:, sl:sl + 1].astype(jnp.float32)
    lo_ref[:, sl + 1:sl + 2] = r1.astype(jnp.bfloat16)
    r2 = r1 - lo_ref[:, sl + 1:sl + 2].astype(jnp.float32)
    lo_ref[:, sl + 2:sl + 3] = r2.astype(jnp.bfloat16)


def _min_kernel(poset_ref, lm4_ref, x_ref, camplus_ref, ids_ref,
                camb_ref, run_ref):
    j = pl.program_id(0)
    i = pl.program_id(1)
    msl = pl.ds(j * _MC, _MC)

    @pl.when(i == 0)
    def _init():
        cam = lax.dot_general(lm4_ref[...], poset_ref[...],
                              (((1,), (0,)), ((), ())),
                              preferred_element_type=jnp.float32)
        camplus_ref[msl, 0:3] = cam
        camb_ref[:, 0:3] = (-2.0 * cam).astype(jnp.bfloat16)
        camb_ref[:, 3:6] = jnp.ones((_MC, 3), jnp.bfloat16)
        y2 = jnp.sum(cam * cam, axis=1, keepdims=True)
        _split3(y2, camb_ref, 6)
        run_ref[...] = jnp.full((_MC, _BS), jnp.float32(3e38), jnp.float32)

    d2 = lax.dot_general(camb_ref[...], x_ref[0],
                         (((1,), (0,)), ((), ())),
                         preferred_element_type=jnp.float32)
    d2 = jnp.maximum(d2, 1e-12)
    key = (lax.bitcast_convert_type(d2, jnp.int32) & jnp.int32(-128)) | i
    keyf = lax.bitcast_convert_type(key, jnp.float32)
    run_ref[...] = jnp.minimum(run_ref[...], keyf)

    @pl.when(i == _NB - 1)
    def _extract():
        keys = lax.bitcast_convert_type(run_ref[...], jnp.int32)
        bi = keys & jnp.int32(127)
        d2t = lax.bitcast_convert_type(keys & jnp.int32(-128), jnp.float32)
        bm = jnp.min(d2t, axis=1, keepdims=True)
        lane = lax.broadcasted_iota(jnp.int32, (_MC, _BS), 1)
        gid = bi * _BS + lane
        cand = jnp.where(d2t == bm, gid, jnp.int32(2 ** 30))
        ids_ref[msl, :] = jnp.min(cand, axis=1, keepdims=True)
        camplus_ref[msl, 3:4] = bm


def _sc_loss_kernel(splat16_ref, camplus_ref, ids_ref, psum_ref, pcnt_ref,
                    idx_v, rows_v, cam_v, acc_v, sem):
    cid = lax.axis_index("c")
    sid = lax.axis_index("s")
    w = sid * _NC + cid
    base = w * _LPW
    pltpu.sync_copy(ids_ref.at[pl.ds(base, _LPW)], idx_v)
    pltpu.async_copy(splat16_ref.at[idx_v], rows_v, sem).wait()
    pltpu.sync_copy(camplus_ref.at[pl.ds(base, _LPW)], cam_v)
    lanes = lax.iota(jnp.int32, 16)
    zero = jnp.zeros((16,), jnp.int32)
    acc_s = jnp.zeros((16,), jnp.float32)
    acc_c = jnp.zeros((16,), jnp.float32)
    for c8 in range(_LPW // _LANES):
        ridx = lanes + c8 * _LANES
        gx = plsc.load_gather(rows_v, [ridx, zero])
        gy = plsc.load_gather(rows_v, [ridx, zero + 1])
        gz = plsc.load_gather(rows_v, [ridx, zero + 2])
        cx = plsc.load_gather(cam_v, [ridx, zero])
        cy = plsc.load_gather(cam_v, [ridx, zero + 1])
        cz = plsc.load_gather(cam_v, [ridx, zero + 2])
        md = plsc.load_gather(cam_v, [ridx, zero + 3])
        dx = gx - cx
        dy = gy - cy
        dz = gz - cz
        sq = dx * dx + dy * dy + dz * dz
        valid = md < (_MAX_DISTANCE * _MAX_DISTANCE)
        acc_s = acc_s + jnp.where(valid, sq, 0.0)
        acc_c = acc_c + jnp.where(valid, 1.0, 0.0)
    acc_v[...] = acc_s
    pltpu.sync_copy(acc_v, psum_ref.at[w])
    acc_v[...] = acc_c
    pltpu.sync_copy(acc_v, pcnt_ref.at[w])


def _finish_kernel(psum_ref, pcnt_ref, out_ref):
    s = jnp.sum(psum_ref[...])
    nv = jnp.sum(pcnt_ref[...])
    out_ref[0, 0] = _LANDMARK_WEIGHT * s / jnp.maximum(3.0 * nv, 1.0)


def kernel(splat_positions, camera_pose, landmarks_3d, landmarks_2d,
           camera_intrinsics):
    del landmarks_2d, camera_intrinsics
    xp = jnp.pad(splat_positions, ((0, _NP - _N_SPLATS), (0, 0)),
                 constant_values=1e6)
    x2 = jnp.sum(xp * xp, axis=1, keepdims=True)
    h1 = lax.reduce_precision(x2, 8, 7)
    r1 = x2 - h1
    h2 = lax.reduce_precision(r1, 8, 7)
    h3 = lax.reduce_precision(r1 - h2, 8, 7)
    x9 = jnp.concatenate(
        [xp, h1, h2, h3, jnp.ones((_NP, 3), jnp.float32)], axis=1
    ).astype(jnp.bfloat16)
    x = x9.reshape(_NB, _BS, 9).transpose(0, 2, 1)
    poset = camera_pose[:3, :].T.astype(jnp.bfloat16)
    lm4 = jnp.concatenate(
        [landmarks_3d, jnp.ones((_M, 1), jnp.float32)], axis=1
    ).astype(jnp.bfloat16)
    splat16 = jnp.pad(splat_positions, ((0, 0), (0, 13)))

    camplus, ids = pl.pallas_call(
        _min_kernel,
        grid=(_NMC, _NB),
        in_specs=[
            pl.BlockSpec((4, 3), lambda j, i: (0, 0)),
            pl.BlockSpec((_MC, 4), lambda j, i: (j, 0)),
            pl.BlockSpec((1, 9, _BS), lambda j, i: (i, 0, 0)),
        ],
        out_specs=[
            pl.BlockSpec((_M, 16), lambda j, i: (0, 0)),
            pl.BlockSpec((_M, 1), lambda j, i: (0, 0)),
        ],
        out_shape=[
            jax.ShapeDtypeStruct((_M, 16), jnp.float32),
            jax.ShapeDtypeStruct((_M, 1), jnp.int32),
        ],
        scratch_shapes=[
            pltpu.VMEM((_MC, 9), jnp.bfloat16),
            pltpu.VMEM((_MC, _BS), jnp.float32),
        ],
    )(poset, lm4, x)

    sc_loss = pl.kernel(
        _sc_loss_kernel,
        out_type=[
            jax.ShapeDtypeStruct((_NW, 16), jnp.float32),
            jax.ShapeDtypeStruct((_NW, 16), jnp.float32),
        ],
        mesh=plsc.VectorSubcoreMesh(core_axis_name="c", subcore_axis_name="s",
                                    num_cores=_NC, num_subcores=_NS),
        compiler_params=pltpu.CompilerParams(needs_layout_passes=False,
                                             use_tc_tiling_on_sc=False),
        scratch_types=[
            pltpu.VMEM((_LPW,), jnp.int32),
            pltpu.VMEM((_LPW, 16), jnp.float32),
            pltpu.VMEM((_LPW, 16), jnp.float32),
            pltpu.VMEM((16,), jnp.float32),
            pltpu.SemaphoreType.DMA,
        ],
    )
    psum, pcnt = sc_loss(splat16, camplus, ids.reshape(_M))

    out = pl.pallas_call(
        _finish_kernel,
        in_specs=[
            pl.BlockSpec(memory_space=pltpu.VMEM),
            pl.BlockSpec(memory_space=pltpu.VMEM),
        ],
        out_specs=pl.BlockSpec(memory_space=pltpu.SMEM),
        out_shape=jax.ShapeDtypeStruct((1, 1), jnp.float32),
    )(psum, pcnt)
    return out[0, 0]
